# Initial kernel scaffold; baseline (speedup 1.0000x reference)
import jax
import jax.numpy as jnp
from jax.experimental import pallas as pl


def kernel(x, edge_index, batch, W_init, b_init, W_np, b_np, Wm, bm, wi, bi, wj, bj, We, be, Wf, bf):
    raise NotImplementedError("write your pallas kernel here")


# speedup vs baseline: 4.0011x; 4.0011x over previous
import functools

import jax
import jax.numpy as jnp
from jax import lax
from jax.experimental import pallas as pl
from jax.experimental.pallas import tpu as pltpu
from jax.experimental.pallas import tpu_sc as plsc

N = 10000
E = 160000
D_IN = 256
H = 512
HF = 256
NCLS = 10
NLAYERS = 4
G = 16

NB = 1000
NBLK = N // NB

C = 4
F = H // C
NC = 2
NS = 16
EPT = E // NS
EB = 80
NBAT = EPT // EB
RB = 624
TAIL = N - NS * RB

_F32 = jnp.float32


def _tc_init_body(x_ref, wini_ref, bini_ref, wm0_ref, bm0_ref, batch_ref,
                  um_ref, psumx_ref, counts_ref):
    i = pl.program_id(0)
    x = x_ref[...]
    u0 = jnp.dot(x, wini_ref[...], preferred_element_type=_F32) + bini_ref[...]
    um = jnp.dot(u0, wm0_ref[...], preferred_element_type=_F32) + bm0_ref[...]
    for c in range(C):
        um_ref[c] = um[:, c * F:(c + 1) * F]
    b = batch_ref[...].reshape(1, NB)
    oh = (lax.broadcasted_iota(jnp.int32, (G, NB), 0) == b).astype(_F32)

    @pl.when(i == 0)
    def _():
        psumx_ref[...] = jnp.zeros_like(psumx_ref)
        counts_ref[...] = jnp.zeros_like(counts_ref)

    psumx_ref[...] += jnp.dot(oh, x, preferred_element_type=_F32)
    counts_ref[...] += jnp.broadcast_to(
        jnp.sum(oh, axis=1, keepdims=True), (G, H))


def _tc_init(x, w_init, b_init, wm0, bm0, batch3):
    return pl.pallas_call(
        _tc_init_body,
        grid=(NBLK,),
        in_specs=[
            pl.BlockSpec((NB, D_IN), lambda i: (i, 0)),
            pl.BlockSpec((D_IN, H), lambda i: (0, 0)),
            pl.BlockSpec((1, H), lambda i: (0, 0)),
            pl.BlockSpec((H, H), lambda i: (0, 0)),
            pl.BlockSpec((1, H), lambda i: (0, 0)),
            pl.BlockSpec((1, 1, NB), lambda i: (i, 0, 0)),
        ],
        out_specs=[
            pl.BlockSpec((C, NB, F), lambda i: (0, i, 0)),
            pl.BlockSpec((G, D_IN), lambda i: (0, 0)),
            pl.BlockSpec((G, H), lambda i: (0, 0)),
        ],
        out_shape=[
            jax.ShapeDtypeStruct((C, N, F), _F32),
            jax.ShapeDtypeStruct((G, D_IN), _F32),
            jax.ShapeDtypeStruct((G, H), _F32),
        ],
        compiler_params=pltpu.CompilerParams(
            dimension_semantics=("arbitrary",)),
    )(x, w_init, b_init, wm0, bm0, batch3)


def _tc_update_body(aggr_ref, um_ref, wi_ref, bi_ref, wj_ref, bj_ref,
                    wn_ref, bn_ref, batch_ref, umn_ref, psum_ref):
    i = pl.program_id(0)
    aggr = jnp.concatenate([aggr_ref[c] for c in range(C)], axis=-1)
    um = jnp.concatenate([um_ref[c] for c in range(C)], axis=-1)
    ai = um * wi_ref[...] + bi_ref[...]
    aj = aggr * wj_ref[...] + bj_ref[...]
    u = aggr + um + ai * aj
    umn = jnp.dot(u, wn_ref[...], preferred_element_type=_F32) + bn_ref[...]
    for c in range(C):
        umn_ref[c] = umn[:, c * F:(c + 1) * F]
    b = batch_ref[...].reshape(1, NB)
    oh = (lax.broadcasted_iota(jnp.int32, (G, NB), 0) == b).astype(_F32)

    @pl.when(i == 0)
    def _():
        psum_ref[...] = jnp.zeros_like(psum_ref)

    psum_ref[...] += jnp.dot(oh, u, preferred_element_type=_F32)


def _tc_update(aggr, um, wi, bi, wj, bj, wn, bn, batch3):
    return pl.pallas_call(
        _tc_update_body,
        grid=(NBLK,),
        in_specs=[
            pl.BlockSpec((C, NB, F), lambda i: (0, i, 0)),
            pl.BlockSpec((C, NB, F), lambda i: (0, i, 0)),
            pl.BlockSpec((1, H), lambda i: (0, 0)),
            pl.BlockSpec((1, H), lambda i: (0, 0)),
            pl.BlockSpec((1, H), lambda i: (0, 0)),
            pl.BlockSpec((1, H), lambda i: (0, 0)),
            pl.BlockSpec((H, H), lambda i: (0, 0)),
            pl.BlockSpec((1, H), lambda i: (0, 0)),
            pl.BlockSpec((1, 1, NB), lambda i: (i, 0, 0)),
        ],
        out_specs=[
            pl.BlockSpec((C, NB, F), lambda i: (0, i, 0)),
            pl.BlockSpec((G, H), lambda i: (0, 0)),
        ],
        out_shape=[
            jax.ShapeDtypeStruct((C, N, F), _F32),
            jax.ShapeDtypeStruct((G, H), _F32),
        ],
        compiler_params=pltpu.CompilerParams(
            dimension_semantics=("arbitrary",)),
    )(aggr, um, wi, bi, wj, bj, wn, bn, batch3)


def _tc_last_body(aggr_ref, um_ref, wi_ref, bi_ref, wj_ref, bj_ref,
                  batch_ref, psum_ref):
    i = pl.program_id(0)
    aggr = jnp.concatenate([aggr_ref[c] for c in range(C)], axis=-1)
    um = jnp.concatenate([um_ref[c] for c in range(C)], axis=-1)
    ai = um * wi_ref[...] + bi_ref[...]
    aj = aggr * wj_ref[...] + bj_ref[...]
    u = aggr + um + ai * aj
    b = batch_ref[...].reshape(1, NB)
    oh = (lax.broadcasted_iota(jnp.int32, (G, NB), 0) == b).astype(_F32)

    @pl.when(i == 0)
    def _():
        psum_ref[...] = jnp.zeros_like(psum_ref)

    psum_ref[...] += jnp.dot(oh, u, preferred_element_type=_F32)


def _tc_last(aggr, um, wi, bi, wj, bj, batch3):
    return pl.pallas_call(
        _tc_last_body,
        grid=(NBLK,),
        in_specs=[
            pl.BlockSpec((C, NB, F), lambda i: (0, i, 0)),
            pl.BlockSpec((C, NB, F), lambda i: (0, i, 0)),
            pl.BlockSpec((1, H), lambda i: (0, 0)),
            pl.BlockSpec((1, H), lambda i: (0, 0)),
            pl.BlockSpec((1, H), lambda i: (0, 0)),
            pl.BlockSpec((1, H), lambda i: (0, 0)),
            pl.BlockSpec((1, 1, NB), lambda i: (i, 0, 0)),
        ],
        out_specs=pl.BlockSpec((G, H), lambda i: (0, 0)),
        out_shape=jax.ShapeDtypeStruct((G, H), _F32),
        compiler_params=pltpu.CompilerParams(
            dimension_semantics=("arbitrary",)),
    )(aggr, um, wi, bi, wj, bj, batch3)


def _tc_head_body(psumx_ref, ps0_ref, ps1_ref, ps2_ref, ps3_ref, counts_ref,
                  wnp_ref, bnp_ref, we_ref, be_ref, wf_ref, bf_ref, out_ref):
    cnt = jnp.maximum(counts_ref[...], 1.0)
    poolx = psumx_ref[...] / cnt[:, :D_IN]
    out = jnp.dot(poolx, wnp_ref[...], preferred_element_type=_F32) + bnp_ref[...]
    for l, ps_ref in enumerate((ps0_ref, ps1_ref, ps2_ref, ps3_ref)):
        pool = ps_ref[...] / cnt
        out += (jnp.dot(pool, we_ref[l], preferred_element_type=_F32)
                + be_ref[l]) * (1.0 / NLAYERS)
    logits = jnp.dot(out, wf_ref[...], preferred_element_type=_F32) + bf_ref[...]
    m = jnp.max(logits, axis=-1, keepdims=True)
    s = logits - m
    lse = jnp.log(jnp.sum(jnp.exp(s), axis=-1, keepdims=True))
    out_ref[...] = s - lse


def _tc_head(psumx, psums, counts, w_np, b_np, we, be, wf, bf):
    return pl.pallas_call(
        _tc_head_body,
        out_shape=jax.ShapeDtypeStruct((G, NCLS), _F32),
    )(psumx, psums[0], psums[1], psums[2], psums[3], counts,
      w_np, b_np, we, be, wf, bf)


def _sc_aggregate(um_flat, src3, dst3, zrows):
    mesh = plsc.VectorSubcoreMesh(core_axis_name="c", subcore_axis_name="s")

    @functools.partial(
        pl.kernel,
        out_type=jax.ShapeDtypeStruct((C * N, F), _F32),
        mesh=mesh,
        scratch_types=[
            pltpu.VMEM((NBAT, EB), jnp.int32),
            pltpu.VMEM((NBAT, EB), jnp.int32),
            pltpu.VMEM((EB, F), _F32),
            pltpu.VMEM_SHARED((N, F), _F32),
            pltpu.SemaphoreType.DMA,
        ],
    )
    def body(um_hbm, src_hbm, dst_hbm, z_hbm, out_hbm,
             src_v, dst_v, rows_v, acc_sh, sem):
        cid = lax.axis_index("c")
        sid = lax.axis_index("s")
        pltpu.sync_copy(src_hbm.at[sid], src_v)
        pltpu.sync_copy(dst_hbm.at[sid], dst_v)

        for c_local in range(NC):
            chunk = cid * NC + c_local
            off = cid * (NC * N) if c_local == 0 else N

            def add_off(k, _):
                r = k // (EB // 16)
                j = k % (EB // 16)
                src_v[r, pl.ds(j * 16, 16)] = (
                    src_v[r, pl.ds(j * 16, 16)] + off)
                return _

            lax.fori_loop(0, (NBAT * EB) // 16, add_off, 0)

            pltpu.sync_copy(z_hbm, acc_sh.at[pl.ds(sid * RB, RB)])

            @pl.when(sid == 0)
            def _():
                pltpu.sync_copy(z_hbm.at[pl.ds(0, TAIL)],
                                acc_sh.at[pl.ds(NS * RB, TAIL)])

            plsc.subcore_barrier()

            def ebody(b, _):
                cp = pltpu.async_copy(um_hbm.at[src_v.at[b]], rows_v, sem)
                cp.wait()
                pltpu.sync_copy(rows_v, acc_sh.at[dst_v.at[b]], add=True)
                return _

            lax.fori_loop(0, NBAT, ebody, 0)
            plsc.subcore_barrier()

            pltpu.sync_copy(
                acc_sh.at[pl.ds(sid * RB, RB)],
                out_hbm.at[pl.ds(chunk * N + sid * RB, RB)])

            @pl.when(sid == 0)
            def _():
                pltpu.sync_copy(
                    acc_sh.at[pl.ds(NS * RB, TAIL)],
                    out_hbm.at[pl.ds(chunk * N + NS * RB, TAIL)])

            plsc.subcore_barrier()

    return body(um_flat, src3, dst3, zrows)


def kernel(x, edge_index, batch, W_init, b_init, W_np, b_np,
           Wm, bm, wi, bi, wj, bj, We, be, Wf, bf):
    src3 = edge_index[0].reshape(NS, NBAT, EB)
    dst3 = edge_index[1].reshape(NS, NBAT, EB)
    batch3 = batch.reshape(NBLK, 1, NB)
    zrows = jnp.zeros((RB, F), _F32)

    b_init2 = b_init.reshape(1, H)
    bm2 = bm.reshape(NLAYERS, 1, H)
    wi2 = wi.reshape(NLAYERS, 1, H)
    bi2 = bi.reshape(NLAYERS, 1, H)
    wj2 = wj.reshape(NLAYERS, 1, H)
    bj2 = bj.reshape(NLAYERS, 1, H)
    be2 = be.reshape(NLAYERS, 1, HF)
    b_np2 = b_np.reshape(1, HF)
    bf2 = bf.reshape(1, NCLS)

    um, psumx, counts = _tc_init(x, W_init, b_init2, Wm[0], bm2[0], batch3)
    psums = []
    for l in range(NLAYERS):
        aggr = _sc_aggregate(um.reshape(C * N, F), src3, dst3, zrows)
        aggr = aggr.reshape(C, N, F)
        if l < NLAYERS - 1:
            um, ps = _tc_update(aggr, um, wi2[l], bi2[l], wj2[l], bj2[l],
                                Wm[l + 1], bm2[l + 1], batch3)
        else:
            ps = _tc_last(aggr, um, wi2[l], bi2[l], wj2[l], bj2[l], batch3)
        psums.append(ps)

    return _tc_head(psumx, psums, counts, W_np, b_np2, We, be2, Wf, bf2)


# speedup vs baseline: 4.9557x; 1.2386x over previous
import functools

import jax
import jax.numpy as jnp
from jax import lax
from jax.experimental import pallas as pl
from jax.experimental.pallas import tpu as pltpu
from jax.experimental.pallas import tpu_sc as plsc

N = 10000
E = 160000
D_IN = 256
H = 512
HF = 256
NCLS = 10
NLAYERS = 4
G = 16

NB = 1000
NBLK = N // NB

C = 4
F = H // C
NC = 2
NS = 16
EPT = E // NS
EB = 80
NBAT = EPT // EB
RB = 624
TAIL = N - NS * RB
ZR = 104

_F32 = jnp.float32


def _tc_init_body(x_ref, wini_ref, bini_ref, wm0_ref, bm0_ref, batch_ref,
                  um_ref, psumx_ref, counts_ref):
    i = pl.program_id(0)
    x = x_ref[...]
    u0 = jnp.dot(x, wini_ref[...], preferred_element_type=_F32) + bini_ref[...]
    um = jnp.dot(u0, wm0_ref[...], preferred_element_type=_F32) + bm0_ref[...]
    for c in range(C):
        um_ref[c] = um[:, c * F:(c + 1) * F]
    b = batch_ref[...].reshape(1, NB)
    oh = (lax.broadcasted_iota(jnp.int32, (G, NB), 0) == b).astype(_F32)

    @pl.when(i == 0)
    def _():
        psumx_ref[...] = jnp.zeros_like(psumx_ref)
        counts_ref[...] = jnp.zeros_like(counts_ref)

    psumx_ref[...] += jnp.dot(oh, x, preferred_element_type=_F32)
    counts_ref[...] += jnp.broadcast_to(
        jnp.sum(oh, axis=1, keepdims=True), (G, H))


def _tc_init(x, w_init, b_init, wm0, bm0, batch3):
    return pl.pallas_call(
        _tc_init_body,
        grid=(NBLK,),
        in_specs=[
            pl.BlockSpec((NB, D_IN), lambda i: (i, 0)),
            pl.BlockSpec((D_IN, H), lambda i: (0, 0)),
            pl.BlockSpec((1, H), lambda i: (0, 0)),
            pl.BlockSpec((H, H), lambda i: (0, 0)),
            pl.BlockSpec((1, H), lambda i: (0, 0)),
            pl.BlockSpec((1, 1, NB), lambda i: (i, 0, 0)),
        ],
        out_specs=[
            pl.BlockSpec((C, NB, F), lambda i: (0, i, 0)),
            pl.BlockSpec((G, D_IN), lambda i: (0, 0)),
            pl.BlockSpec((G, H), lambda i: (0, 0)),
        ],
        out_shape=[
            jax.ShapeDtypeStruct((C, N, F), _F32),
            jax.ShapeDtypeStruct((G, D_IN), _F32),
            jax.ShapeDtypeStruct((G, H), _F32),
        ],
        compiler_params=pltpu.CompilerParams(
            dimension_semantics=("arbitrary",)),
    )(x, w_init, b_init, wm0, bm0, batch3)


def _tc_update_body(aggr_ref, um_ref, wi_ref, bi_ref, wj_ref, bj_ref,
                    wn_ref, bn_ref, batch_ref, umn_ref, psum_ref):
    i = pl.program_id(0)
    aggr = jnp.concatenate([aggr_ref[c] for c in range(C)], axis=-1)
    um = jnp.concatenate([um_ref[c] for c in range(C)], axis=-1)
    ai = um * wi_ref[...] + bi_ref[...]
    aj = aggr * wj_ref[...] + bj_ref[...]
    u = aggr + um + ai * aj
    umn = jnp.dot(u, wn_ref[...], preferred_element_type=_F32) + bn_ref[...]
    for c in range(C):
        umn_ref[c] = umn[:, c * F:(c + 1) * F]
    b = batch_ref[...].reshape(1, NB)
    oh = (lax.broadcasted_iota(jnp.int32, (G, NB), 0) == b).astype(_F32)

    @pl.when(i == 0)
    def _():
        psum_ref[...] = jnp.zeros_like(psum_ref)

    psum_ref[...] += jnp.dot(oh, u, preferred_element_type=_F32)


def _tc_update(aggr, um, wi, bi, wj, bj, wn, bn, batch3):
    return pl.pallas_call(
        _tc_update_body,
        grid=(NBLK,),
        in_specs=[
            pl.BlockSpec((C, NB, F), lambda i: (0, i, 0)),
            pl.BlockSpec((C, NB, F), lambda i: (0, i, 0)),
            pl.BlockSpec((1, H), lambda i: (0, 0)),
            pl.BlockSpec((1, H), lambda i: (0, 0)),
            pl.BlockSpec((1, H), lambda i: (0, 0)),
            pl.BlockSpec((1, H), lambda i: (0, 0)),
            pl.BlockSpec((H, H), lambda i: (0, 0)),
            pl.BlockSpec((1, H), lambda i: (0, 0)),
            pl.BlockSpec((1, 1, NB), lambda i: (i, 0, 0)),
        ],
        out_specs=[
            pl.BlockSpec((C, NB, F), lambda i: (0, i, 0)),
            pl.BlockSpec((G, H), lambda i: (0, 0)),
        ],
        out_shape=[
            jax.ShapeDtypeStruct((C, N, F), _F32),
            jax.ShapeDtypeStruct((G, H), _F32),
        ],
        compiler_params=pltpu.CompilerParams(
            dimension_semantics=("arbitrary",)),
    )(aggr, um, wi, bi, wj, bj, wn, bn, batch3)


def _tc_last_body(aggr_ref, um_ref, wi_ref, bi_ref, wj_ref, bj_ref,
                  batch_ref, psum_ref):
    i = pl.program_id(0)
    aggr = jnp.concatenate([aggr_ref[c] for c in range(C)], axis=-1)
    um = jnp.concatenate([um_ref[c] for c in range(C)], axis=-1)
    ai = um * wi_ref[...] + bi_ref[...]
    aj = aggr * wj_ref[...] + bj_ref[...]
    u = aggr + um + ai * aj
    b = batch_ref[...].reshape(1, NB)
    oh = (lax.broadcasted_iota(jnp.int32, (G, NB), 0) == b).astype(_F32)

    @pl.when(i == 0)
    def _():
        psum_ref[...] = jnp.zeros_like(psum_ref)

    psum_ref[...] += jnp.dot(oh, u, preferred_element_type=_F32)


def _tc_last(aggr, um, wi, bi, wj, bj, batch3):
    return pl.pallas_call(
        _tc_last_body,
        grid=(NBLK,),
        in_specs=[
            pl.BlockSpec((C, NB, F), lambda i: (0, i, 0)),
            pl.BlockSpec((C, NB, F), lambda i: (0, i, 0)),
            pl.BlockSpec((1, H), lambda i: (0, 0)),
            pl.BlockSpec((1, H), lambda i: (0, 0)),
            pl.BlockSpec((1, H), lambda i: (0, 0)),
            pl.BlockSpec((1, H), lambda i: (0, 0)),
            pl.BlockSpec((1, 1, NB), lambda i: (i, 0, 0)),
        ],
        out_specs=pl.BlockSpec((G, H), lambda i: (0, 0)),
        out_shape=jax.ShapeDtypeStruct((G, H), _F32),
        compiler_params=pltpu.CompilerParams(
            dimension_semantics=("arbitrary",)),
    )(aggr, um, wi, bi, wj, bj, batch3)


def _tc_head_body(psumx_ref, ps0_ref, ps1_ref, ps2_ref, ps3_ref, counts_ref,
                  wnp_ref, bnp_ref, we_ref, be_ref, wf_ref, bf_ref, out_ref):
    cnt = jnp.maximum(counts_ref[...], 1.0)
    poolx = psumx_ref[...] / cnt[:, :D_IN]
    out = jnp.dot(poolx, wnp_ref[...], preferred_element_type=_F32) + bnp_ref[...]
    for l, ps_ref in enumerate((ps0_ref, ps1_ref, ps2_ref, ps3_ref)):
        pool = ps_ref[...] / cnt
        out += (jnp.dot(pool, we_ref[l], preferred_element_type=_F32)
                + be_ref[l]) * (1.0 / NLAYERS)
    logits = jnp.dot(out, wf_ref[...], preferred_element_type=_F32) + bf_ref[...]
    m = jnp.max(logits, axis=-1, keepdims=True)
    s = logits - m
    lse = jnp.log(jnp.sum(jnp.exp(s), axis=-1, keepdims=True))
    out_ref[...] = s - lse


def _tc_head(psumx, psums, counts, w_np, b_np, we, be, wf, bf):
    return pl.pallas_call(
        _tc_head_body,
        out_shape=jax.ShapeDtypeStruct((G, NCLS), _F32),
    )(psumx, psums[0], psums[1], psums[2], psums[3], counts,
      w_np, b_np, we, be, wf, bf)


def _sc_aggregate(um_flat, src2, dst3, zrows):
    mesh = plsc.VectorSubcoreMesh(core_axis_name="c", subcore_axis_name="s")

    @functools.partial(
        pl.kernel,
        out_type=jax.ShapeDtypeStruct((C * N, F), _F32),
        mesh=mesh,
        scratch_types=[
            pltpu.VMEM((EPT,), jnp.int32),
            pltpu.VMEM((NBAT, EB), jnp.int32),
            pltpu.VMEM((EB, F), _F32),
            pltpu.VMEM((EB, F), _F32),
            pltpu.VMEM_SHARED((N, F), _F32),
            pltpu.SemaphoreType.DMA,
            pltpu.SemaphoreType.DMA,
            pltpu.SemaphoreType.DMA,
            pltpu.SemaphoreType.DMA,
        ],
    )
    def body(um_hbm, src_hbm, dst_hbm, z_hbm, out_hbm,
             src_v, dst_v, rows_a, rows_b, acc_sh, gsa, gsb, ssa, ssb):
        cid = lax.axis_index("c")
        sid = lax.axis_index("s")
        pltpu.sync_copy(src_hbm.at[sid], src_v)
        pltpu.sync_copy(dst_hbm.at[sid], dst_v)

        for c_local in range(NC):
            chunk = cid * NC + c_local
            off = cid * (NC * N) if c_local == 0 else N

            def add_off(k, carry):
                src_v[pl.ds(k * 16, 16)] = src_v[pl.ds(k * 16, 16)] + off
                return carry

            lax.fori_loop(0, EPT // 16, add_off, 0)

            for z in range(RB // ZR):
                pltpu.sync_copy(
                    z_hbm, acc_sh.at[pl.ds(sid * RB + z * ZR, ZR)])

            @pl.when(sid == 0)
            def _():
                pltpu.sync_copy(z_hbm.at[pl.ds(0, TAIL)],
                                acc_sh.at[pl.ds(NS * RB, TAIL)])

            plsc.subcore_barrier()

            def sidx(b):
                return src_v.at[pl.ds(pl.multiple_of(b * EB, 8), EB)]

            pltpu.async_copy(um_hbm.at[sidx(0)], rows_a, gsa)

            def pair(i, carry):
                g = i * 2
                pltpu.make_async_copy(
                    um_hbm.at[sidx(g)], rows_a, gsa).wait()

                @pl.when(g > 0)
                def _():
                    pltpu.make_async_copy(
                        rows_b, acc_sh.at[dst_v.at[g]], ssb).wait()

                pltpu.async_copy(um_hbm.at[sidx(g + 1)], rows_b, gsb)
                pltpu.async_copy(rows_a, acc_sh.at[dst_v.at[g]], ssa,
                                 add=True)
                pltpu.make_async_copy(
                    um_hbm.at[sidx(g + 1)], rows_b, gsb).wait()
                pltpu.make_async_copy(
                    rows_a, acc_sh.at[dst_v.at[g]], ssa).wait()

                @pl.when(g + 2 < NBAT)
                def _():
                    pltpu.async_copy(um_hbm.at[sidx(g + 2)], rows_a, gsa)

                pltpu.async_copy(rows_b, acc_sh.at[dst_v.at[g + 1]], ssb,
                                 add=True)
                return carry

            lax.fori_loop(0, NBAT // 2, pair, 0)
            pltpu.make_async_copy(
                um_hbm.at[sidx(NBAT - 1)], rows_a, gsa).wait()
            pltpu.make_async_copy(
                rows_b, acc_sh.at[dst_v.at[NBAT - 2]], ssb).wait()
            pltpu.async_copy(rows_a, acc_sh.at[dst_v.at[NBAT - 1]], ssa,
                             add=True)
            pltpu.make_async_copy(
                rows_a, acc_sh.at[dst_v.at[NBAT - 1]], ssa).wait()
            plsc.subcore_barrier()

            pltpu.sync_copy(
                acc_sh.at[pl.ds(sid * RB, RB)],
                out_hbm.at[pl.ds(chunk * N + sid * RB, RB)])

            @pl.when(sid == 0)
            def _():
                pltpu.sync_copy(
                    acc_sh.at[pl.ds(NS * RB, TAIL)],
                    out_hbm.at[pl.ds(chunk * N + NS * RB, TAIL)])

            plsc.subcore_barrier()

    return body(um_flat, src2, dst3, zrows)


def kernel(x, edge_index, batch, W_init, b_init, W_np, b_np,
           Wm, bm, wi, bi, wj, bj, We, be, Wf, bf):
    src2 = edge_index[0].reshape(NS, EPT)
    dst3 = edge_index[1].reshape(NS, NBAT, EB)
    batch3 = batch.reshape(NBLK, 1, NB)
    zrows = jnp.zeros((ZR, F), _F32)

    b_init2 = b_init.reshape(1, H)
    bm2 = bm.reshape(NLAYERS, 1, H)
    wi2 = wi.reshape(NLAYERS, 1, H)
    bi2 = bi.reshape(NLAYERS, 1, H)
    wj2 = wj.reshape(NLAYERS, 1, H)
    bj2 = bj.reshape(NLAYERS, 1, H)
    be2 = be.reshape(NLAYERS, 1, HF)
    b_np2 = b_np.reshape(1, HF)
    bf2 = bf.reshape(1, NCLS)

    um, psumx, counts = _tc_init(x, W_init, b_init2, Wm[0], bm2[0], batch3)
    psums = []
    for l in range(NLAYERS):
        aggr = _sc_aggregate(um.reshape(C * N, F), src2, dst3, zrows)
        aggr = aggr.reshape(C, N, F)
        if l < NLAYERS - 1:
            um, ps = _tc_update(aggr, um, wi2[l], bi2[l], wj2[l], bj2[l],
                                Wm[l + 1], bm2[l + 1], batch3)
        else:
            ps = _tc_last(aggr, um, wi2[l], bi2[l], wj2[l], bj2[l], batch3)
        psums.append(ps)

    return _tc_head(psumx, psums, counts, W_np, b_np2, We, be2, Wf, bf2)
